# Initial kernel scaffold; baseline (speedup 1.0000x reference)
import functools

import jax
import jax.numpy as jnp
from jax import lax
from jax.experimental import pallas as pl
from jax.experimental.pallas import tpu as pltpu

B, T, DS = 4, 8192, 1024
N, DR = 256, 1024
HID = 128
K = 32
TCHUNK = 512
NSTEPS = T // TCHUNK
RCHUNK = (B * N) // NSTEPS

_HI = jax.lax.Precision.HIGHEST


def _body(s_ref, r_ref, wst_ref, wct_ref, w1t_ref, b1_ref, w2_ref, b2_ref,
          out_ref, acc_ref, rw_ref):
    t = pl.program_id(0)

    @pl.when(t == 0)
    def _init():
        acc_ref[...] = jnp.zeros_like(acc_ref)

    acc_ref[...] += jnp.sum(s_ref[...], axis=1)

    rw_ref[pl.ds(t * RCHUNK, RCHUNK), :] = lax.dot_general(
        r_ref[...], wst_ref[...], (((1,), (0,)), ((), ())),
        preferred_element_type=jnp.float32, precision=_HI)

    @pl.when(t == NSTEPS - 1)
    def _tail():
        context = acc_ref[...] * (1.0 / T)
        ctx = lax.dot_general(context, wct_ref[...],
                              (((1,), (0,)), ((), ())),
                              preferred_element_type=jnp.float32,
                              precision=_HI)
        b2 = b2_ref[0, 0]
        for b in range(B):
            rwb = rw_ref[pl.ds(b * N, N), :]
            h = jnp.tanh(rwb + ctx[b:b + 1, :])
            hid = jnp.maximum(
                lax.dot_general(h, w1t_ref[...], (((1,), (0,)), ((), ())),
                                preferred_element_type=jnp.float32,
                                precision=_HI) + b1_ref[...], 0.0)
            sc_col = lax.dot_general(hid, w2_ref[...],
                                     (((1,), (1,)), ((), ())),
                                     preferred_element_type=jnp.float32,
                                     precision=_HI) + b2
            sc_row = lax.dot_general(w2_ref[...], hid,
                                     (((1,), (1,)), ((), ())),
                                     preferred_element_type=jnp.float32,
                                     precision=_HI) + b2
            col = lax.broadcast_in_dim(sc_col, (N, N), (0, 1))
            row = lax.broadcast_in_dim(sc_row, (N, N), (0, 1))
            ii = lax.broadcasted_iota(jnp.int32, (N, N), 0)
            jj = lax.broadcasted_iota(jnp.int32, (N, N), 1)
            beats = (col > row) | ((col == row) & (ii < jj))
            rank = jnp.sum(beats.astype(jnp.int32), axis=0, keepdims=True)
            out_ref[pl.ds(b, 1), :] = (rank < K).astype(jnp.float32)


def _allocate(s, r2, wst, wct, w1t, b1r, w2, b2r, interpret=False):
    return pl.pallas_call(
        _body,
        grid=(NSTEPS,),
        in_specs=[
            pl.BlockSpec((B, TCHUNK, DS), lambda t: (0, t, 0)),
            pl.BlockSpec((RCHUNK, DR), lambda t: (t, 0)),
            pl.BlockSpec((DR, DR), lambda t: (0, 0)),
            pl.BlockSpec((DS, DR), lambda t: (0, 0)),
            pl.BlockSpec((DR, HID), lambda t: (0, 0)),
            pl.BlockSpec((1, HID), lambda t: (0, 0)),
            pl.BlockSpec((1, HID), lambda t: (0, 0)),
            pl.BlockSpec((1, 1), lambda t: (0, 0)),
        ],
        out_specs=pl.BlockSpec((B, N), lambda t: (0, 0)),
        out_shape=jax.ShapeDtypeStruct((B, N), jnp.float32),
        scratch_shapes=[
            pltpu.VMEM((B, DS), jnp.float32),
            pltpu.VMEM((B * N, DR), jnp.float32),
        ],
        interpret=interpret,
    )(s, r2, wst, wct, w1t, b1r, w2, b2r)


@jax.jit
def kernel(s, r, Wc, Ws, W1, b1, W2, b2):
    mask = _allocate(s, r.reshape(B * N, DR), Ws.T, Wc.T, W1.T,
                     b1.reshape(1, HID), W2, b2.reshape(1, 1))
    return mask[..., None]


# speedup vs baseline: 1.0640x; 1.0640x over previous
import functools

import jax
import jax.numpy as jnp
from jax import lax
from jax.experimental import pallas as pl
from jax.experimental.pallas import tpu as pltpu

B, T, DS = 4, 8192, 1024
N, DR = 256, 1024
HID = 128
K = 32
TCHUNK = 512
NSTEPS = T // TCHUNK
RCHUNK = (B * N) // NSTEPS


def _body(s_ref, r_ref, wst_ref, wct_ref, w1t_ref, b1_ref, w2_ref, b2_ref,
          out_ref, acc_ref, rw_ref):
    t = pl.program_id(0)

    @pl.when(t == 0)
    def _init():
        acc_ref[...] = jnp.zeros_like(acc_ref)

    acc_ref[...] += jnp.sum(s_ref[...], axis=1)

    rw_ref[pl.ds(t * RCHUNK, RCHUNK), :] = lax.dot_general(
        r_ref[...], wst_ref[...], (((1,), (0,)), ((), ())),
        preferred_element_type=jnp.float32)

    @pl.when(t == NSTEPS - 1)
    def _tail():
        context = acc_ref[...] * (1.0 / T)
        ctx = lax.dot_general(context, wct_ref[...],
                              (((1,), (0,)), ((), ())),
                              preferred_element_type=jnp.float32)
        b2 = b2_ref[0, 0]
        for b in range(B):
            rwb = rw_ref[pl.ds(b * N, N), :]
            h = jnp.tanh(rwb + ctx[b:b + 1, :])
            hid = jnp.maximum(
                lax.dot_general(h, w1t_ref[...], (((1,), (0,)), ((), ())),
                                preferred_element_type=jnp.float32) + b1_ref[...], 0.0)
            sc_col = jnp.sum(hid * w2_ref[...], axis=1,
                             keepdims=True) + b2
            col = lax.broadcast_in_dim(sc_col, (N, N), (0, 1))
            row = lax.transpose(col, (1, 0))
            ii = lax.broadcasted_iota(jnp.int32, (N, N), 0)
            jj = lax.broadcasted_iota(jnp.int32, (N, N), 1)
            beats = (col > row) | ((col == row) & (ii < jj))
            rank = jnp.sum(beats.astype(jnp.int32), axis=0, keepdims=True)
            out_ref[pl.ds(b, 1), :] = (rank < K).astype(jnp.float32)


def _allocate(s, r2, wst, wct, w1t, b1r, w2, b2r, interpret=False):
    return pl.pallas_call(
        _body,
        grid=(NSTEPS,),
        in_specs=[
            pl.BlockSpec((B, TCHUNK, DS), lambda t: (0, t, 0)),
            pl.BlockSpec((RCHUNK, DR), lambda t: (t, 0)),
            pl.BlockSpec((DR, DR), lambda t: (0, 0)),
            pl.BlockSpec((DS, DR), lambda t: (0, 0)),
            pl.BlockSpec((DR, HID), lambda t: (0, 0)),
            pl.BlockSpec((1, HID), lambda t: (0, 0)),
            pl.BlockSpec((1, HID), lambda t: (0, 0)),
            pl.BlockSpec((1, 1), lambda t: (0, 0)),
        ],
        out_specs=pl.BlockSpec((B, N), lambda t: (0, 0)),
        out_shape=jax.ShapeDtypeStruct((B, N), jnp.float32),
        scratch_shapes=[
            pltpu.VMEM((B, DS), jnp.float32),
            pltpu.VMEM((B * N, DR), jnp.float32),
        ],
        interpret=interpret,
    )(s, r2, wst, wct, w1t, b1r, w2, b2r)


@jax.jit
def kernel(s, r, Wc, Ws, W1, b1, W2, b2):
    mask = _allocate(s, r.reshape(B * N, DR), Ws.T, Wc.T, W1.T,
                     b1.reshape(1, HID), W2, b2.reshape(1, 1))
    return mask[..., None]
